# baseline (speedup 1.0000x reference)
import functools

import jax
import jax.numpy as jnp
from jax import lax
from jax.experimental import pallas as pl
from jax.experimental.pallas import tpu as pltpu
from jax.experimental.pallas import tpu_sc as plsc

H_IMG = 128
W_IMG = 128
D_MODEL = 256
N_HEADS = 8
N_POINTS = 4
LQ = H_IMG * W_IMG
HD = D_MODEL // N_HEADS
NK = 4 * N_HEADS * N_POINTS

N_CORES = 2
N_SUBCORES = 16
NW = N_CORES * N_SUBCORES
QPW = LQ // NW
CQ = 16
NCHUNKS = QPW // CQ

BS = 512
GRID = LQ // BS


def _prep_body(qry, qpos, inflat, refx, refy, wval, bval, woffx, woffy,
               boffx, boffy, wattn, battn, val_out, idx_out, wgt_out):
    q = qry[0] + qpos[0]
    val_out[...] = (
        jnp.dot(inflat[0], wval[...], preferred_element_type=jnp.float32)
        + bval[...]
    ).astype(jnp.bfloat16)
    offx = jnp.dot(q, woffx[...], preferred_element_type=jnp.float32) + boffx[...]
    offy = jnp.dot(q, woffy[...], preferred_element_type=jnp.float32) + boffy[...]
    logits = jnp.dot(q, wattn[...], preferred_element_type=jnp.float32) + battn[...]
    e = jnp.exp(logits)
    gi = lax.broadcasted_iota(jnp.int32, (32, 32), 0) // N_POINTS
    gj = lax.broadcasted_iota(jnp.int32, (32, 32), 1) // N_POINTS
    gmat = (gi == gj).astype(jnp.float32)
    s = jnp.dot(e, gmat, preferred_element_type=jnp.float32)
    attnw = e / s

    x = refx[...] * W_IMG + offx - 0.5
    y = refy[...] * H_IMG + offy - 0.5
    x0 = jnp.floor(x)
    y0 = jnp.floor(y)
    fx = x - x0
    fy = y - y0

    tile4 = lambda a: jnp.concatenate([a, a, a, a], axis=1)
    x0t, fxt = tile4(x0), tile4(fx)
    y0t, fyt = tile4(y0), tile4(fy)
    at = tile4(attnw)
    lane = lax.broadcasted_iota(jnp.int32, (BS, NK), 1)
    gx = (lane >= 64).astype(jnp.float32)
    gy = ((lane // 32) % 2).astype(jnp.float32)
    xq = x0t + gx
    yq = y0t + gy
    selx = jnp.where(gx > 0.0, fxt, 1.0 - fxt)
    sely = jnp.where(gy > 0.0, fyt, 1.0 - fyt)
    valid = ((xq >= 0.0) & (xq <= W_IMG - 1.0)
             & (yq >= 0.0) & (yq <= H_IMG - 1.0))
    wgt_out[...] = at * selx * sely * valid.astype(jnp.float32)
    xc = jnp.clip(xq, 0.0, W_IMG - 1.0)
    yc = jnp.clip(yq, 0.0, H_IMG - 1.0)
    mt = ((lane % 32) // N_POINTS).astype(jnp.float32)
    idx_out[...] = ((yc * W_IMG + xc) * N_HEADS + mt).astype(jnp.int32)


_prep = pl.pallas_call(
    _prep_body,
    grid=(GRID,),
    in_specs=[
        pl.BlockSpec((1, BS, D_MODEL), lambda i: (0, i, 0)),
        pl.BlockSpec((1, BS, D_MODEL), lambda i: (0, i, 0)),
        pl.BlockSpec((1, BS, D_MODEL), lambda i: (0, i, 0)),
        pl.BlockSpec((BS, 1), lambda i: (i, 0)),
        pl.BlockSpec((BS, 1), lambda i: (i, 0)),
        pl.BlockSpec((D_MODEL, D_MODEL), lambda i: (0, 0)),
        pl.BlockSpec((1, D_MODEL), lambda i: (0, 0)),
        pl.BlockSpec((D_MODEL, N_HEADS * N_POINTS), lambda i: (0, 0)),
        pl.BlockSpec((D_MODEL, N_HEADS * N_POINTS), lambda i: (0, 0)),
        pl.BlockSpec((1, N_HEADS * N_POINTS), lambda i: (0, 0)),
        pl.BlockSpec((1, N_HEADS * N_POINTS), lambda i: (0, 0)),
        pl.BlockSpec((D_MODEL, N_HEADS * N_POINTS), lambda i: (0, 0)),
        pl.BlockSpec((1, N_HEADS * N_POINTS), lambda i: (0, 0)),
    ],
    out_specs=[
        pl.BlockSpec((BS, D_MODEL), lambda i: (i, 0)),
        pl.BlockSpec((BS, NK), lambda i: (i, 0)),
        pl.BlockSpec((BS, NK), lambda i: (i, 0)),
    ],
    out_shape=[
        jax.ShapeDtypeStruct((LQ, D_MODEL), jnp.bfloat16),
        jax.ShapeDtypeStruct((LQ, NK), jnp.int32),
        jax.ShapeDtypeStruct((LQ, NK), jnp.float32),
    ],
)


def _proj_body(x, w, b, o):
    o[...] = jnp.dot(x[...], w[...], preferred_element_type=jnp.float32) + b[...]


PBS = 2048

_proj = pl.pallas_call(
    _proj_body,
    grid=(LQ // PBS,),
    in_specs=[
        pl.BlockSpec((PBS, D_MODEL), lambda i: (i, 0)),
        pl.BlockSpec((D_MODEL, D_MODEL), lambda i: (0, 0)),
        pl.BlockSpec((1, D_MODEL), lambda i: (0, 0)),
    ],
    out_specs=pl.BlockSpec((PBS, D_MODEL), lambda i: (i, 0)),
    out_shape=jax.ShapeDtypeStruct((LQ, D_MODEL), jnp.float32),
)


def _sc_body(table, idxs, wgts, out_hbm,
             idx_v0, idx_v1, w_v0, w_v1, rows0, rows1, out0, out1,
             gi0, gi1, gw0, gw1, gg0, gg1, go0, go1):
    idx_vs = (idx_v0, idx_v1)
    w_vs = (w_v0, w_v1)
    rows = (rows0, rows1)
    outs = (out0, out1)
    si = (gi0, gi1)
    sw = (gw0, gw1)
    sg = (gg0, gg1)
    so = (go0, go1)
    wid = lax.axis_index("s") * N_CORES + lax.axis_index("c")
    base = wid * QPW

    def issue_gathers(b, t):
        for j in range(CQ):
            pltpu.async_copy(
                table.at[idx_vs[b].at[j]],
                rows[b].at[pl.ds(j * NK, NK)],
                sg[b],
            )

    def drain(src, dst, sem):
        pltpu.make_async_copy(src, dst, sem).wait()

    def compute_chunk(b):
        w_v = w_vs[b]
        rows_v = rows[b]
        out_v = outs[b]

        @plsc.parallel_loop(0, CQ, step=1, unroll=2)
        def per_q(j):
            r0 = j * NK
            for mh in range(N_HEADS):
                acc0 = jnp.zeros((16,), jnp.float32)
                acc1 = jnp.zeros((16,), jnp.float32)
                for c4 in range(4):
                    wgrp = w_v[j, pl.ds(c4 * 32 + (mh // 4) * 16, 16)]
                    gacc = None
                    for p in range(N_POINTS):
                        k = c4 * 32 + mh * N_POINTS + p
                        wsc = wgrp[(mh % 4) * N_POINTS + p]
                        wf = lax.broadcast(wsc, (16,))
                        wb = plsc.pack(wf, wf, format=plsc.PackFormat.INTERLEAVED)
                        term = rows_v[r0 + k, :] * wb
                        gacc = term if gacc is None else gacc + term
                    lo, hi = plsc.unpack(gacc,
                                         format=plsc.PackFormat.INTERLEAVED)
                    acc0 = acc0 + lo
                    acc1 = acc1 + hi
                out_v[j, pl.ds(mh * HD, HD)] = plsc.pack(
                    acc0, acc1, format=plsc.PackFormat.INTERLEAVED)

    pltpu.sync_copy(idxs.at[pl.ds(base, CQ)], idx_v0)
    pltpu.sync_copy(wgts.at[pl.ds(base, CQ)], w_v0)
    pltpu.sync_copy(idxs.at[pl.ds(base + CQ, CQ)], idx_v1)
    pltpu.sync_copy(wgts.at[pl.ds(base + CQ, CQ)], w_v1)
    issue_gathers(0, 0)

    def step(t, b):
        b1 = 1 - b
        tn = t + 1
        @pl.when(tn < NCHUNKS)
        def _():
            @pl.when(tn >= 2)
            def _():
                drain(idxs.at[pl.ds(base, CQ)], idx_vs[b1], si[b1])
            issue_gathers(b1, tn)

        drain(table.at[pl.ds(0, CQ * NK)], rows[b], sg[b])
        @pl.when(t + 2 < NCHUNKS)
        def _():
            pltpu.async_copy(idxs.at[pl.ds(base + (t + 2) * CQ, CQ)],
                             idx_vs[b], si[b])
        @pl.when(t >= 2)
        def _():
            drain(outs[b], out_hbm.at[pl.ds(base, CQ)], so[b])
            drain(wgts.at[pl.ds(base, CQ)], w_vs[b], sw[b])

        compute_chunk(b)

        @pl.when(t + 2 < NCHUNKS)
        def _():
            pltpu.async_copy(wgts.at[pl.ds(base + (t + 2) * CQ, CQ)],
                             w_vs[b], sw[b])

        pltpu.async_copy(outs[b], out_hbm.at[pl.ds(base + t * CQ, CQ)], so[b])

    def pair(g, carry):
        step(2 * g, 0)
        step(2 * g + 1, 1)
        return carry

    lax.fori_loop(0, NCHUNKS // 2, pair, 0)
    drain(out0, out_hbm.at[pl.ds(base, CQ)], go0)
    drain(out1, out_hbm.at[pl.ds(base, CQ)], go1)


@functools.cache
def _sc_sample_call():
    return pl.kernel(
        _sc_body,
        out_type=jax.ShapeDtypeStruct((LQ, D_MODEL), jnp.bfloat16),
        mesh=plsc.VectorSubcoreMesh(
            core_axis_name="c", subcore_axis_name="s",
            num_cores=N_CORES, num_subcores=N_SUBCORES,
        ),
        scratch_types=[
            pltpu.VMEM((CQ, NK), jnp.int32),
            pltpu.VMEM((CQ, NK), jnp.int32),
            pltpu.VMEM((CQ, NK), jnp.float32),
            pltpu.VMEM((CQ, NK), jnp.float32),
            pltpu.VMEM((CQ * NK, HD), jnp.bfloat16),
            pltpu.VMEM((CQ * NK, HD), jnp.bfloat16),
            pltpu.VMEM((CQ, D_MODEL), jnp.bfloat16),
            pltpu.VMEM((CQ, D_MODEL), jnp.bfloat16),
            pltpu.SemaphoreType.DMA,
            pltpu.SemaphoreType.DMA,
            pltpu.SemaphoreType.DMA,
            pltpu.SemaphoreType.DMA,
            pltpu.SemaphoreType.DMA,
            pltpu.SemaphoreType.DMA,
            pltpu.SemaphoreType.DMA,
            pltpu.SemaphoreType.DMA,
        ],
        compiler_params=pltpu.CompilerParams(
            use_tc_tiling_on_sc=False, needs_layout_passes=False,
        ),
    )


def kernel(query, query_pos, reference_points, input_flatten,
           W_off, b_off, W_attn, b_attn, W_val, b_val, W_out, b_out):
    rp = reference_points.reshape(LQ, 2)
    refx = rp[:, 0:1]
    refy = rp[:, 1:2]
    woff = W_off.reshape(D_MODEL, N_HEADS, N_POINTS, 2)
    woffx = woff[..., 0].reshape(D_MODEL, N_HEADS * N_POINTS)
    woffy = woff[..., 1].reshape(D_MODEL, N_HEADS * N_POINTS)
    boff = b_off.reshape(N_HEADS, N_POINTS, 2)
    boffx = boff[..., 0].reshape(1, N_HEADS * N_POINTS)
    boffy = boff[..., 1].reshape(1, N_HEADS * N_POINTS)
    battn2 = b_attn.reshape(1, N_HEADS * N_POINTS)
    bval2 = b_val.reshape(1, D_MODEL)
    bout2 = b_out.reshape(1, D_MODEL)

    value, idx, wgt = _prep(query, query_pos, input_flatten, refx, refy,
                            W_val, bval2, woffx, woffy, boffx, boffy,
                            W_attn, battn2)
    table = value.reshape(LQ * N_HEADS, HD)
    sampled = _sc_sample_call()(table, idx, wgt)
    out = _proj(sampled, W_out, bout2)
    return out.reshape(1, LQ, D_MODEL)


# speedup vs baseline: 1.0458x; 1.0458x over previous
import functools

import jax
import jax.numpy as jnp
from jax import lax
from jax.experimental import pallas as pl
from jax.experimental.pallas import tpu as pltpu
from jax.experimental.pallas import tpu_sc as plsc

H_IMG = 128
W_IMG = 128
D_MODEL = 256
N_HEADS = 8
N_POINTS = 4
LQ = H_IMG * W_IMG
HD = D_MODEL // N_HEADS
NK = 4 * N_HEADS * N_POINTS

N_CORES = 2
N_SUBCORES = 16
NW = N_CORES * N_SUBCORES
QPW = LQ // NW
CQ = 16
NCHUNKS = QPW // CQ

BS = 512
GRID = LQ // BS


def _prep_body(qry, qpos, inflat, refxy, wval, bval, woffx, woffy,
               boffx, boffy, wattn, battn, val_out, idx_out, wgt_out):
    q = qry[0] + qpos[0]
    refx = refxy[:, 0:1]
    refy = refxy[:, 1:2]
    val_out[...] = (
        jnp.dot(inflat[0], wval[...], preferred_element_type=jnp.float32)
        + bval[...]
    ).astype(jnp.bfloat16)
    offx = jnp.dot(q, woffx[...], preferred_element_type=jnp.float32) + boffx[...]
    offy = jnp.dot(q, woffy[...], preferred_element_type=jnp.float32) + boffy[...]
    logits = jnp.dot(q, wattn[...], preferred_element_type=jnp.float32) + battn[...]
    e = jnp.exp(logits)
    gi = lax.broadcasted_iota(jnp.int32, (32, 32), 0) // N_POINTS
    gj = lax.broadcasted_iota(jnp.int32, (32, 32), 1) // N_POINTS
    gmat = (gi == gj).astype(jnp.float32)
    s = jnp.dot(e, gmat, preferred_element_type=jnp.float32)
    attnw = e / s

    x = refx * W_IMG + offx - 0.5
    y = refy * H_IMG + offy - 0.5
    x0 = jnp.floor(x)
    y0 = jnp.floor(y)
    fx = x - x0
    fy = y - y0

    tile4 = lambda a: jnp.concatenate([a, a, a, a], axis=1)
    x0t, fxt = tile4(x0), tile4(fx)
    y0t, fyt = tile4(y0), tile4(fy)
    at = tile4(attnw)
    lane = lax.broadcasted_iota(jnp.int32, (BS, NK), 1)
    gx = (lane >= 64).astype(jnp.float32)
    gy = ((lane // 32) % 2).astype(jnp.float32)
    xq = x0t + gx
    yq = y0t + gy
    selx = jnp.where(gx > 0.0, fxt, 1.0 - fxt)
    sely = jnp.where(gy > 0.0, fyt, 1.0 - fyt)
    valid = ((xq >= 0.0) & (xq <= W_IMG - 1.0)
             & (yq >= 0.0) & (yq <= H_IMG - 1.0))
    wgt_out[...] = at * selx * sely * valid.astype(jnp.float32)
    xc = jnp.clip(xq, 0.0, W_IMG - 1.0)
    yc = jnp.clip(yq, 0.0, H_IMG - 1.0)
    mt = ((lane % 32) // N_POINTS).astype(jnp.float32)
    idx_out[...] = ((yc * W_IMG + xc) * N_HEADS + mt).astype(jnp.int32)


_prep = pl.pallas_call(
    _prep_body,
    grid=(GRID,),
    in_specs=[
        pl.BlockSpec((1, BS, D_MODEL), lambda i: (0, i, 0)),
        pl.BlockSpec((1, BS, D_MODEL), lambda i: (0, i, 0)),
        pl.BlockSpec((1, BS, D_MODEL), lambda i: (0, i, 0)),
        pl.BlockSpec((BS, 2), lambda i: (i, 0)),
        pl.BlockSpec((D_MODEL, D_MODEL), lambda i: (0, 0)),
        pl.BlockSpec((1, D_MODEL), lambda i: (0, 0)),
        pl.BlockSpec((D_MODEL, N_HEADS * N_POINTS), lambda i: (0, 0)),
        pl.BlockSpec((D_MODEL, N_HEADS * N_POINTS), lambda i: (0, 0)),
        pl.BlockSpec((1, N_HEADS * N_POINTS), lambda i: (0, 0)),
        pl.BlockSpec((1, N_HEADS * N_POINTS), lambda i: (0, 0)),
        pl.BlockSpec((D_MODEL, N_HEADS * N_POINTS), lambda i: (0, 0)),
        pl.BlockSpec((1, N_HEADS * N_POINTS), lambda i: (0, 0)),
    ],
    out_specs=[
        pl.BlockSpec((BS, D_MODEL), lambda i: (i, 0)),
        pl.BlockSpec((BS, NK), lambda i: (i, 0)),
        pl.BlockSpec((BS, NK), lambda i: (i, 0)),
    ],
    out_shape=[
        jax.ShapeDtypeStruct((LQ, D_MODEL), jnp.bfloat16),
        jax.ShapeDtypeStruct((LQ, NK), jnp.int32),
        jax.ShapeDtypeStruct((LQ, NK), jnp.float32),
    ],
)


def _proj_body(x, w, b, o):
    x32 = x[...].astype(jnp.float32)
    o[...] = jnp.dot(x32, w[...], preferred_element_type=jnp.float32) + b[...]


PBS = 2048

_proj = pl.pallas_call(
    _proj_body,
    grid=(LQ // PBS,),
    in_specs=[
        pl.BlockSpec((PBS, D_MODEL), lambda i: (i, 0)),
        pl.BlockSpec((D_MODEL, D_MODEL), lambda i: (0, 0)),
        pl.BlockSpec((1, D_MODEL), lambda i: (0, 0)),
    ],
    out_specs=pl.BlockSpec((PBS, D_MODEL), lambda i: (i, 0)),
    out_shape=jax.ShapeDtypeStruct((LQ, D_MODEL), jnp.float32),
)


def _sc_body(table, idxs, wgts, out_hbm,
             idx_v0, idx_v1, w_v0, w_v1, rows0, rows1, out0, out1,
             gi0, gi1, gw0, gw1, gg0, gg1, go0, go1):
    idx_vs = (idx_v0, idx_v1)
    w_vs = (w_v0, w_v1)
    rows = (rows0, rows1)
    outs = (out0, out1)
    si = (gi0, gi1)
    sw = (gw0, gw1)
    sg = (gg0, gg1)
    so = (go0, go1)
    wid = lax.axis_index("s") * N_CORES + lax.axis_index("c")
    base = wid * QPW

    def issue_gathers(b, t):
        for j in range(CQ):
            pltpu.async_copy(
                table.at[idx_vs[b].at[j]],
                rows[b].at[pl.ds(j * NK, NK)],
                sg[b],
            )

    def drain(src, dst, sem):
        pltpu.make_async_copy(src, dst, sem).wait()

    def compute_chunk(b):
        w_v = w_vs[b]
        rows_v = rows[b]
        out_v = outs[b]

        @plsc.parallel_loop(0, CQ, step=1, unroll=2)
        def per_q(j):
            r0 = j * NK
            for mh in range(N_HEADS):
                acc0 = jnp.zeros((16,), jnp.float32)
                acc1 = jnp.zeros((16,), jnp.float32)
                for c4 in range(4):
                    wgrp = w_v[j, pl.ds(c4 * 32 + (mh // 4) * 16, 16)]
                    gacc = None
                    for p in range(N_POINTS):
                        k = c4 * 32 + mh * N_POINTS + p
                        wsc = wgrp[(mh % 4) * N_POINTS + p]
                        wf = lax.broadcast(wsc, (16,))
                        wb = plsc.pack(wf, wf, format=plsc.PackFormat.INTERLEAVED)
                        term = rows_v[r0 + k, :] * wb
                        gacc = term if gacc is None else gacc + term
                    lo, hi = plsc.unpack(gacc,
                                         format=plsc.PackFormat.INTERLEAVED)
                    acc0 = acc0 + lo
                    acc1 = acc1 + hi
                out_v[j, pl.ds(mh * HD, HD)] = plsc.pack(
                    acc0, acc1, format=plsc.PackFormat.INTERLEAVED)

    pltpu.sync_copy(idxs.at[pl.ds(base, CQ)], idx_v0)
    pltpu.sync_copy(wgts.at[pl.ds(base, CQ)], w_v0)
    pltpu.sync_copy(idxs.at[pl.ds(base + CQ, CQ)], idx_v1)
    pltpu.sync_copy(wgts.at[pl.ds(base + CQ, CQ)], w_v1)
    issue_gathers(0, 0)

    def step(t, b):
        b1 = 1 - b
        tn = t + 1
        @pl.when(tn < NCHUNKS)
        def _():
            @pl.when(tn >= 2)
            def _():
                drain(idxs.at[pl.ds(base, CQ)], idx_vs[b1], si[b1])
            issue_gathers(b1, tn)

        drain(table.at[pl.ds(0, CQ * NK)], rows[b], sg[b])
        @pl.when(t + 2 < NCHUNKS)
        def _():
            pltpu.async_copy(idxs.at[pl.ds(base + (t + 2) * CQ, CQ)],
                             idx_vs[b], si[b])
        @pl.when(t >= 2)
        def _():
            drain(outs[b], out_hbm.at[pl.ds(base, CQ)], so[b])
            drain(wgts.at[pl.ds(base, CQ)], w_vs[b], sw[b])

        compute_chunk(b)

        @pl.when(t + 2 < NCHUNKS)
        def _():
            pltpu.async_copy(wgts.at[pl.ds(base + (t + 2) * CQ, CQ)],
                             w_vs[b], sw[b])

        pltpu.async_copy(outs[b], out_hbm.at[pl.ds(base + t * CQ, CQ)], so[b])

    def pair(g, carry):
        step(2 * g, 0)
        step(2 * g + 1, 1)
        return carry

    lax.fori_loop(0, NCHUNKS // 2, pair, 0)
    drain(out0, out_hbm.at[pl.ds(base, CQ)], go0)
    drain(out1, out_hbm.at[pl.ds(base, CQ)], go1)


@functools.cache
def _sc_sample_call():
    return pl.kernel(
        _sc_body,
        out_type=jax.ShapeDtypeStruct((LQ, D_MODEL), jnp.bfloat16),
        mesh=plsc.VectorSubcoreMesh(
            core_axis_name="c", subcore_axis_name="s",
            num_cores=N_CORES, num_subcores=N_SUBCORES,
        ),
        scratch_types=[
            pltpu.VMEM((CQ, NK), jnp.int32),
            pltpu.VMEM((CQ, NK), jnp.int32),
            pltpu.VMEM((CQ, NK), jnp.float32),
            pltpu.VMEM((CQ, NK), jnp.float32),
            pltpu.VMEM((CQ * NK, HD), jnp.bfloat16),
            pltpu.VMEM((CQ * NK, HD), jnp.bfloat16),
            pltpu.VMEM((CQ, D_MODEL), jnp.bfloat16),
            pltpu.VMEM((CQ, D_MODEL), jnp.bfloat16),
            pltpu.SemaphoreType.DMA,
            pltpu.SemaphoreType.DMA,
            pltpu.SemaphoreType.DMA,
            pltpu.SemaphoreType.DMA,
            pltpu.SemaphoreType.DMA,
            pltpu.SemaphoreType.DMA,
            pltpu.SemaphoreType.DMA,
            pltpu.SemaphoreType.DMA,
        ],
        compiler_params=pltpu.CompilerParams(
            use_tc_tiling_on_sc=False, needs_layout_passes=False,
        ),
    )


def kernel(query, query_pos, reference_points, input_flatten,
           W_off, b_off, W_attn, b_attn, W_val, b_val, W_out, b_out):
    rp = reference_points.reshape(LQ, 2)
    woff = W_off.reshape(D_MODEL, N_HEADS, N_POINTS, 2)
    woffx = woff[..., 0].reshape(D_MODEL, N_HEADS * N_POINTS)
    woffy = woff[..., 1].reshape(D_MODEL, N_HEADS * N_POINTS)
    boff = b_off.reshape(N_HEADS, N_POINTS, 2)
    boffx = boff[..., 0].reshape(1, N_HEADS * N_POINTS)
    boffy = boff[..., 1].reshape(1, N_HEADS * N_POINTS)
    battn2 = b_attn.reshape(1, N_HEADS * N_POINTS)
    bval2 = b_val.reshape(1, D_MODEL)
    bout2 = b_out.reshape(1, D_MODEL)

    value, idx, wgt = _prep(query, query_pos, input_flatten, rp,
                            W_val, bval2, woffx, woffy, boffx, boffy,
                            W_attn, battn2)
    table = value.reshape(LQ * N_HEADS, HD)
    sampled = _sc_sample_call()(table, idx, wgt)
    out = _proj(sampled, W_out, bout2)
    return out.reshape(1, LQ, D_MODEL)
